# baseline (speedup 1.0000x reference)
import functools

import jax
import jax.numpy as jnp
from jax import lax
from jax.experimental import pallas as pl
from jax.experimental.pallas import tpu as pltpu
from jax.experimental.pallas import tpu_sc as plsc

B = 1024
N = 100000
D = 768
H = 128
K = 64
CHUNK = 2048
G = 16
NCHUNKS = (N + CHUNK - 1) // CHUNK
NPAD = NCHUNKS * CHUNK
NG = NPAD // G
GPC = CHUNK // G
SW = 128
SUPER = NPAD // SW
SPG = SW // G
BB = 256
RB = 256
RB4 = 128
NEG = float("-inf")
BIG = 2**30


def _score_body(q_ref, nq_ref, c_ref, nc_ref, w1_ref, b1_ref, w2_ref, b2_ref,
                s_ref, gm_ref, qn_ref, cn_ref, nu_ref):
    j = pl.program_id(0)
    i = pl.program_id(1)

    @pl.when((j == 0) & (i == 0))
    def _():
        q = q_ref[...]
        qn_ref[...] = q / jnp.maximum(nq_ref[...], 1e-12)

    @pl.when(i == 0)
    def _():
        c = c_ref[...]
        row = lax.broadcasted_iota(jnp.int32, (CHUNK, 1), 0) + j * CHUNK
        c = jnp.where(row < N, c, 0.0)
        n = jnp.where(row < N, nc_ref[...], 1.0)
        cn_ref[...] = c / jnp.maximum(n, 1e-12)
        h = jax.lax.dot_general(c, w1_ref[...], (((1,), (0,)), ((), ())),
                                preferred_element_type=jnp.float32)
        h = jnp.maximum(h + b1_ref[...], 0.0)
        z = jax.lax.dot_general(h, w2_ref[...], (((1,), (0,)), ((), ())),
                                preferred_element_type=jnp.float32)
        z = z + b2_ref[...]
        nu_ref[...] = jax.nn.sigmoid(z).reshape(1, CHUNK)

    qs = qn_ref[pl.ds(i * BB, BB), :]
    sim = jax.lax.dot_general(qs, cn_ref[...], (((1,), (1,)), ((), ())),
                              preferred_element_type=jnp.float32)
    s = sim * nu_ref[...]

    col = lax.broadcasted_iota(jnp.int32, (BB, CHUNK), 1) + j * CHUNK
    s = jnp.where(col < N, s, NEG)
    for t in range(CHUNK // SW):
        s_ref[:, t, :] = s[:, t * SW:(t + 1) * SW]
    gm_ref[...] = jnp.max(s.reshape(BB, GPC, G), axis=2)


def _score(q, nq, c, nc, w1, b1t, w2, b2t):
    return pl.pallas_call(
        _score_body,
        grid=(NCHUNKS, B // BB),
        in_specs=[
            pl.BlockSpec((B, D), lambda j, i: (0, 0)),
            pl.BlockSpec((B, 1), lambda j, i: (0, 0)),
            pl.BlockSpec((CHUNK, D), lambda j, i: (j, 0)),
            pl.BlockSpec((CHUNK, 1), lambda j, i: (j, 0)),
            pl.BlockSpec((D, H), lambda j, i: (0, 0)),
            pl.BlockSpec((1, H), lambda j, i: (0, 0)),
            pl.BlockSpec((H, 1), lambda j, i: (0, 0)),
            pl.BlockSpec((1, 1), lambda j, i: (0, 0)),
        ],
        out_specs=[
            pl.BlockSpec((BB, CHUNK // SW, SW), lambda j, i: (i, j, 0)),
            pl.BlockSpec((BB, GPC), lambda j, i: (i, j)),
        ],
        out_shape=[
            jax.ShapeDtypeStruct((B, SUPER, SW), jnp.float32),
            jax.ShapeDtypeStruct((B, NG), jnp.float32),
        ],
        scratch_shapes=[
            pltpu.VMEM((B, D), jnp.float32),
            pltpu.VMEM((CHUNK, D), jnp.float32),
            pltpu.VMEM((1, CHUNK), jnp.float32),
        ],
    )(q, nq, c, nc, w1, b1t, w2, b2t)


def _groupsel_body(gm_ref, gid_ref, flat_ref, v_ref):
    i = pl.program_id(0)
    v_ref[...] = gm_ref[...]
    col = lax.broadcasted_iota(jnp.int32, (RB, NG), 1)
    col64 = lax.broadcasted_iota(jnp.int32, (RB, K), 1)

    def body(k, acc):
        v = v_ref[...]
        m = jnp.max(v, axis=1, keepdims=True)
        sel = jnp.min(jnp.where(v == m, col, BIG), axis=1, keepdims=True)
        v_ref[...] = jnp.where(col == sel, NEG, v)
        return jnp.where(col64 == k, jnp.broadcast_to(sel, (RB, K)), acc)

    gid = lax.fori_loop(0, K, body, jnp.zeros((RB, K), jnp.int32))
    gid_ref[...] = gid
    rowg = lax.broadcasted_iota(jnp.int32, (RB, K), 0) + i * RB
    flat_ref[...] = rowg * SUPER + gid // SPG


def _groupsel(gm):
    return pl.pallas_call(
        _groupsel_body,
        grid=(B // RB,),
        in_specs=[pl.BlockSpec((RB, NG), lambda i: (i, 0))],
        out_specs=[
            pl.BlockSpec((RB, K), lambda i: (i, 0)),
            pl.BlockSpec((RB, K), lambda i: (i, 0)),
        ],
        out_shape=[
            jax.ShapeDtypeStruct((B, K), jnp.int32),
            jax.ShapeDtypeStruct((B, K), jnp.int32),
        ],
        scratch_shapes=[pltpu.VMEM((RB, NG), jnp.float32)],
    )(gm)


def _finaltopk_body(cand_ref, gid_ref, scores_ref, idx_ref, v_ref):
    gid = gid_ref[...]
    gidw = jnp.broadcast_to(gid[:, :, None], (RB4, K, SW)).reshape(RB4, K * SW)
    colw = lax.broadcasted_iota(jnp.int32, (RB4, K * SW), 1)
    valid = (colw % SW) // G == gidw % SPG
    masked = jnp.where(valid, cand_ref[...], NEG)
    comp = jnp.max(masked.reshape(RB4, K, SPG, G), axis=2).reshape(RB4, K * G)

    gidb = jnp.broadcast_to(gid[:, :, None], (RB4, K, G)).reshape(RB4, K * G)
    col = lax.broadcasted_iota(jnp.int32, (RB4, K * G), 1)
    gcol = gidb * G + (col % G)
    col64 = lax.broadcasted_iota(jnp.int32, (RB4, K), 1)
    v_ref[...] = comp

    def body(k, carry):
        accs, acci = carry
        v = v_ref[...]
        m = jnp.max(v, axis=1, keepdims=True)
        sel = jnp.min(jnp.where(v == m, gcol, BIG), axis=1, keepdims=True)
        v_ref[...] = jnp.where(gcol == sel, NEG, v)
        accs = jnp.where(col64 == k, jnp.broadcast_to(m, (RB4, K)), accs)
        acci = jnp.where(col64 == k, jnp.broadcast_to(sel, (RB4, K)), acci)
        return accs, acci

    accs, acci = lax.fori_loop(
        0, K, body,
        (jnp.zeros((RB4, K), jnp.float32), jnp.zeros((RB4, K), jnp.int32)))
    scores_ref[...] = accs
    idx_ref[...] = acci


def _finaltopk(cand, gid):
    return pl.pallas_call(
        _finaltopk_body,
        grid=(B // RB4,),
        in_specs=[
            pl.BlockSpec((RB4, K * SW), lambda i: (i, 0)),
            pl.BlockSpec((RB4, K), lambda i: (i, 0)),
        ],
        out_specs=[
            pl.BlockSpec((RB4, K), lambda i: (i, 0)),
            pl.BlockSpec((RB4, K), lambda i: (i, 0)),
        ],
        out_shape=[
            jax.ShapeDtypeStruct((B, K), jnp.float32),
            jax.ShapeDtypeStruct((B, K), jnp.int32),
        ],
        scratch_shapes=[pltpu.VMEM((RB4, K * G), jnp.float32)],
    )(cand, gid)


def _make_sc_gather(rows_total, row_w, chunk):
    info = plsc.get_sparse_core_info()
    nw = info.num_cores * info.num_subcores
    per_w = rows_total // nw
    n_iter = per_w // chunk
    mesh = plsc.VectorSubcoreMesh(core_axis_name="c", subcore_axis_name="s")

    @functools.partial(
        pl.kernel,
        mesh=mesh,
        out_type=jax.ShapeDtypeStruct((rows_total, row_w), jnp.float32),
        scratch_types=[
            pltpu.VMEM((chunk,), jnp.int32),
            pltpu.VMEM((chunk, row_w), jnp.float32),
            pltpu.SemaphoreType.DMA,
        ],
    )
    def gather(table_hbm, idx_hbm, out_hbm, idx_v, rows_v, sem):
        wid = lax.axis_index("s") * info.num_cores + lax.axis_index("c")
        base = pl.multiple_of(wid * per_w, chunk)

        def body(i, _):
            b = pl.multiple_of(base + i * chunk, chunk)
            pltpu.sync_copy(idx_hbm.at[pl.ds(b, chunk)], idx_v)
            pltpu.async_copy(table_hbm.at[idx_v], rows_v, sem).wait()
            pltpu.sync_copy(rows_v, out_hbm.at[pl.ds(b, chunk)])
            return 0

        lax.fori_loop(0, n_iter, body, 0)

    return gather


def kernel(query_embedding, corpus_embeddings, W1, b1, W2, b2):
    gather_super = _make_sc_gather(B * K, SW, 128)
    gather_rows = _make_sc_gather(B * K, D, 128)
    b1t = b1.reshape(1, H)
    b2t = b2.reshape(1, 1)
    nq = jnp.linalg.norm(query_embedding, axis=1, keepdims=True)
    nc = jnp.linalg.norm(corpus_embeddings, axis=1, keepdims=True)

    s, gm = _score(query_embedding, nq, corpus_embeddings, nc,
                   W1, b1t, W2, b2t)
    gid, flat = _groupsel(gm)
    cand = gather_super(s.reshape(B * SUPER, SW), flat.reshape(B * K))
    scores, idx = _finaltopk(cand.reshape(B, K * SW), gid)
    retrieved = gather_rows(corpus_embeddings, idx.reshape(B * K))
    return retrieved.reshape(B, K, D), scores, idx


# speedup vs baseline: 1.0445x; 1.0445x over previous
import functools

import jax
import jax.numpy as jnp
from jax import lax
from jax.experimental import pallas as pl
from jax.experimental.pallas import tpu as pltpu
from jax.experimental.pallas import tpu_sc as plsc

B = 1024
N = 100000
D = 768
H = 128
K = 64
CHUNK = 2048
G = 16
NCHUNKS = (N + CHUNK - 1) // CHUNK
NPAD = NCHUNKS * CHUNK
NG = NPAD // G
GPC = CHUNK // G
SW = 128
SUPER = NPAD // SW
SPG = SW // G
BB = 256
RB = 256
RB4 = 128
NEG = float("-inf")
BIG = 2**30


def _score_body(q_ref, nq_ref, c_ref, nc_ref, w1_ref, b1_ref, w2_ref, b2_ref,
                s_ref, gm_ref, qn_ref, cn_ref, nu_ref):
    j = pl.program_id(0)
    i = pl.program_id(1)

    @pl.when((j == 0) & (i == 0))
    def _():
        q = q_ref[...]
        qn_ref[...] = q / jnp.maximum(nq_ref[...], 1e-12)

    @pl.when(i == 0)
    def _():
        c = c_ref[...]
        row = lax.broadcasted_iota(jnp.int32, (CHUNK, 1), 0) + j * CHUNK
        c = jnp.where(row < N, c, 0.0)
        n = jnp.where(row < N, nc_ref[...], 1.0)
        cn_ref[...] = c / jnp.maximum(n, 1e-12)
        h = jax.lax.dot_general(c, w1_ref[...], (((1,), (0,)), ((), ())),
                                preferred_element_type=jnp.float32)
        h = jnp.maximum(h + b1_ref[...], 0.0)
        z = jax.lax.dot_general(h, w2_ref[...], (((1,), (0,)), ((), ())),
                                preferred_element_type=jnp.float32)
        z = z + b2_ref[...]
        nu_ref[...] = jax.nn.sigmoid(z).reshape(1, CHUNK)

    qs = qn_ref[pl.ds(i * BB, BB), :]
    sim = jax.lax.dot_general(qs, cn_ref[...], (((1,), (1,)), ((), ())),
                              preferred_element_type=jnp.float32)
    s = sim * nu_ref[...]

    col = lax.broadcasted_iota(jnp.int32, (BB, CHUNK), 1) + j * CHUNK
    s = jnp.where(col < N, s, NEG)
    s_ref[...] = s.reshape(BB, CHUNK // SW, SW)
    gm_ref[...] = jnp.max(s.reshape(BB, GPC, G), axis=2)


def _score(q, nq, c, nc, w1, b1t, w2, b2t):
    return pl.pallas_call(
        _score_body,
        grid=(NCHUNKS, B // BB),
        in_specs=[
            pl.BlockSpec((B, D), lambda j, i: (0, 0)),
            pl.BlockSpec((B, 1), lambda j, i: (0, 0)),
            pl.BlockSpec((CHUNK, D), lambda j, i: (j, 0)),
            pl.BlockSpec((CHUNK, 1), lambda j, i: (j, 0)),
            pl.BlockSpec((D, H), lambda j, i: (0, 0)),
            pl.BlockSpec((1, H), lambda j, i: (0, 0)),
            pl.BlockSpec((H, 1), lambda j, i: (0, 0)),
            pl.BlockSpec((1, 1), lambda j, i: (0, 0)),
        ],
        out_specs=[
            pl.BlockSpec((BB, CHUNK // SW, SW), lambda j, i: (i, j, 0)),
            pl.BlockSpec((BB, GPC), lambda j, i: (i, j)),
        ],
        out_shape=[
            jax.ShapeDtypeStruct((B, SUPER, SW), jnp.float32),
            jax.ShapeDtypeStruct((B, NG), jnp.float32),
        ],
        scratch_shapes=[
            pltpu.VMEM((B, D), jnp.float32),
            pltpu.VMEM((CHUNK, D), jnp.float32),
            pltpu.VMEM((1, CHUNK), jnp.float32),
        ],
    )(q, nq, c, nc, w1, b1t, w2, b2t)


def _groupsel_body(gm_ref, gid_ref, flat_ref, v_ref):
    i = pl.program_id(0)
    v_ref[...] = gm_ref[...]
    col = lax.broadcasted_iota(jnp.int32, (RB, NG), 1)
    col64 = lax.broadcasted_iota(jnp.int32, (RB, K), 1)

    def body(k, acc):
        v = v_ref[...]
        m = jnp.max(v, axis=1, keepdims=True)
        sel = jnp.min(jnp.where(v == m, col, BIG), axis=1, keepdims=True)
        v_ref[...] = jnp.where(col == sel, NEG, v)
        return jnp.where(col64 == k, jnp.broadcast_to(sel, (RB, K)), acc)

    gid = lax.fori_loop(0, K, body, jnp.zeros((RB, K), jnp.int32))
    gid_ref[...] = gid
    rowg = lax.broadcasted_iota(jnp.int32, (RB, K), 0) + i * RB
    flat_ref[...] = rowg * SUPER + gid // SPG


def _groupsel(gm):
    return pl.pallas_call(
        _groupsel_body,
        grid=(B // RB,),
        in_specs=[pl.BlockSpec((RB, NG), lambda i: (i, 0))],
        out_specs=[
            pl.BlockSpec((RB, K), lambda i: (i, 0)),
            pl.BlockSpec((RB, K), lambda i: (i, 0)),
        ],
        out_shape=[
            jax.ShapeDtypeStruct((B, K), jnp.int32),
            jax.ShapeDtypeStruct((B, K), jnp.int32),
        ],
        scratch_shapes=[pltpu.VMEM((RB, NG), jnp.float32)],
    )(gm)


def _finaltopk_body(cand_ref, gid_ref, scores_ref, idx_ref, v_ref):
    gid = gid_ref[...]
    gidw = jnp.broadcast_to(gid[:, :, None], (RB4, K, SW)).reshape(RB4, K * SW)
    colw = lax.broadcasted_iota(jnp.int32, (RB4, K * SW), 1)
    valid = (colw % SW) // G == gidw % SPG
    masked = jnp.where(valid, cand_ref[...], NEG)
    comp = jnp.max(masked.reshape(RB4, K, SPG, G), axis=2).reshape(RB4, K * G)

    gidb = jnp.broadcast_to(gid[:, :, None], (RB4, K, G)).reshape(RB4, K * G)
    col = lax.broadcasted_iota(jnp.int32, (RB4, K * G), 1)
    gcol = gidb * G + (col % G)
    col64 = lax.broadcasted_iota(jnp.int32, (RB4, K), 1)
    v_ref[...] = comp

    def body(k, carry):
        accs, acci = carry
        v = v_ref[...]
        m = jnp.max(v, axis=1, keepdims=True)
        sel = jnp.min(jnp.where(v == m, gcol, BIG), axis=1, keepdims=True)
        v_ref[...] = jnp.where(gcol == sel, NEG, v)
        accs = jnp.where(col64 == k, jnp.broadcast_to(m, (RB4, K)), accs)
        acci = jnp.where(col64 == k, jnp.broadcast_to(sel, (RB4, K)), acci)
        return accs, acci

    accs, acci = lax.fori_loop(
        0, K, body,
        (jnp.zeros((RB4, K), jnp.float32), jnp.zeros((RB4, K), jnp.int32)))
    scores_ref[...] = accs
    idx_ref[...] = acci


def _finaltopk(cand, gid):
    return pl.pallas_call(
        _finaltopk_body,
        grid=(B // RB4,),
        in_specs=[
            pl.BlockSpec((RB4, K * SW), lambda i: (i, 0)),
            pl.BlockSpec((RB4, K), lambda i: (i, 0)),
        ],
        out_specs=[
            pl.BlockSpec((RB4, K), lambda i: (i, 0)),
            pl.BlockSpec((RB4, K), lambda i: (i, 0)),
        ],
        out_shape=[
            jax.ShapeDtypeStruct((B, K), jnp.float32),
            jax.ShapeDtypeStruct((B, K), jnp.int32),
        ],
        scratch_shapes=[pltpu.VMEM((RB4, K * G), jnp.float32)],
    )(cand, gid)


def _make_sc_gather(rows_total, row_w, chunk):
    info = plsc.get_sparse_core_info()
    nw = info.num_cores * info.num_subcores
    per_w = rows_total // nw
    n_iter = per_w // chunk
    mesh = plsc.VectorSubcoreMesh(core_axis_name="c", subcore_axis_name="s")

    @functools.partial(
        pl.kernel,
        mesh=mesh,
        out_type=jax.ShapeDtypeStruct((rows_total, row_w), jnp.float32),
        scratch_types=[
            pltpu.VMEM((chunk,), jnp.int32),
            pltpu.VMEM((chunk, row_w), jnp.float32),
            pltpu.SemaphoreType.DMA,
        ],
    )
    def gather(table_hbm, idx_hbm, out_hbm, idx_v, rows_v, sem):
        wid = lax.axis_index("s") * info.num_cores + lax.axis_index("c")
        base = pl.multiple_of(wid * per_w, chunk)

        def body(i, _):
            b = pl.multiple_of(base + i * chunk, chunk)
            pltpu.sync_copy(idx_hbm.at[pl.ds(b, chunk)], idx_v)
            pltpu.async_copy(table_hbm.at[idx_v], rows_v, sem).wait()
            pltpu.sync_copy(rows_v, out_hbm.at[pl.ds(b, chunk)])
            return 0

        lax.fori_loop(0, n_iter, body, 0)

    return gather


def kernel(query_embedding, corpus_embeddings, W1, b1, W2, b2):
    gather_super = _make_sc_gather(B * K, SW, 128)
    gather_rows = _make_sc_gather(B * K, D, 128)
    b1t = b1.reshape(1, H)
    b2t = b2.reshape(1, 1)
    nq = jnp.linalg.norm(query_embedding, axis=1, keepdims=True)
    nc = jnp.linalg.norm(corpus_embeddings, axis=1, keepdims=True)

    s, gm = _score(query_embedding, nq, corpus_embeddings, nc,
                   W1, b1t, W2, b2t)
    gid, flat = _groupsel(gm)
    cand = gather_super(s.reshape(B * SUPER, SW), flat.reshape(B * K))
    scores, idx = _finaltopk(cand.reshape(B, K * SW), gid)
    retrieved = gather_rows(corpus_embeddings, idx.reshape(B * K))
    return retrieved.reshape(B, K, D), scores, idx
